# baseline (speedup 1.0000x reference)
import functools

import jax
import jax.numpy as jnp
from jax import lax
from jax.experimental import pallas as pl
from jax.experimental.pallas import tpu as pltpu
from jax.experimental.pallas import tpu_sc as plsc

DIM = 128
CTX = 2048
B = 4
TOK = B * CTX
NC, NS, LANES = 2, 16, 16
NW = NC * NS
NPOS = CTX // NW

CHUNKS = (
    (0, 0, 32), (0, 32, 32),
    (1, 0, 64),
    (2, 0, 64),
    (3, 0, 32), (3, 32, 32),
)
NCHUNK = len(CHUNKS)
MAXCH = 64


@functools.partial(
    pl.kernel,
    out_type=jax.ShapeDtypeStruct((TOK, DIM), jnp.float32),
    mesh=plsc.VectorSubcoreMesh(core_axis_name="c", subcore_axis_name="s"),
    scratch_types=[
        pltpu.VMEM((LANES, 2 * NPOS), jnp.int32),
        pltpu.VMEM((NPOS, DIM), jnp.float32),
        pltpu.VMEM((B * NPOS, DIM), jnp.float32),
    ]
    + [pltpu.SemaphoreType.DMA] * NCHUNK
    + [
        pltpu.SemaphoreType.DMA,
        pltpu.SemaphoreType.DMA,
    ],
)
def _embed_sc(x_hbm, table_hbm, pos_hbm, out_hbm,
              idx_v, pos_v, rows_v, *sems):
    gsems, psem, ssem = sems[:NCHUNK], sems[NCHUNK], sems[NCHUNK + 1]
    wid = lax.axis_index("s") * NC + lax.axis_index("c")
    pbase = wid * NPOS

    pcp = pltpu.async_copy(pos_hbm.at[pl.ds(pbase, NPOS)], pos_v, psem)
    xrows = jnp.minimum(
        lax.iota(jnp.int32, LANES) * (NW // 2) + wid // 2,
        B * NW // 2 - 1,
    )
    pltpu.async_copy(x_hbm.at[xrows], idx_v, gsems[0]).wait()
    col = lax.rem(wid, 2) * NPOS

    gathers = []
    rofs_list = []
    rofs = 0
    for k, (b, pofs, n) in enumerate(CHUNKS):
        rofs_list.append(rofs)
        gathers.append(
            pltpu.async_copy(
                table_hbm.at[idx_v.at[b, pl.ds(col + pofs, n)]],
                rows_v.at[pl.ds(rofs, n)],
                gsems[k],
            )
        )
        rofs += n
    pcp.wait()

    stores = []
    for k, (b, pofs, n) in enumerate(CHUNKS):
        gathers[k].wait()
        rofs = rofs_list[k]

        def row_body(i, carry):
            r = i * 2
            for u in range(2):
                for c in range(DIM // LANES):
                    s = pl.ds(c * LANES, LANES)
                    plsc.addupdate(
                        rows_v.at[rofs + r + u, s], pos_v[pofs + r + u, s]
                    )
            return carry

        lax.fori_loop(0, n // 2, row_body, 0)
        stores.append(
            pltpu.async_copy(
                rows_v.at[pl.ds(rofs, n)],
                out_hbm.at[pl.ds(b * CTX + pbase + pofs, n)],
                ssem,
            )
        )

    for cp in stores:
        cp.wait()


def kernel(x, table, pos_emb):
    x_rows = x.reshape(B * NW // 2, 2 * NPOS).astype(jnp.int32)
    out = _embed_sc(x_rows, table, pos_emb)
    return out.reshape(B, CTX, DIM)


# speedup vs baseline: 1.6515x; 1.6515x over previous
import functools

import jax
import jax.numpy as jnp
from jax import lax
from jax.experimental import pallas as pl
from jax.experimental.pallas import tpu as pltpu
from jax.experimental.pallas import tpu_sc as plsc

DIM = 128
CTX = 2048
B = 4
TOK = B * CTX
NC, NS, LANES = 2, 16, 16
NW = NC * NS
NPOS = CTX // NW

CHUNKS = (
    (0, 0, 32), (0, 32, 32),
    (1, 0, 64),
    (2, 0, 64),
    (3, 0, 32), (3, 32, 32),
)
NCHUNK = len(CHUNKS)
MAXCH = 64


@functools.partial(
    pl.kernel,
    out_type=jax.ShapeDtypeStruct((TOK, DIM), jnp.float32),
    mesh=plsc.VectorSubcoreMesh(core_axis_name="c", subcore_axis_name="s"),
    scratch_types=[
        pltpu.VMEM((NCHUNK, MAXCH), jnp.int32),
        pltpu.VMEM((NPOS, DIM), jnp.float32),
        pltpu.VMEM((B * NPOS, DIM), jnp.float32),
    ]
    + [pltpu.SemaphoreType.DMA] * NCHUNK
    + [
        pltpu.SemaphoreType.DMA,
        pltpu.SemaphoreType.DMA,
    ],
)
def _embed_sc(x_hbm, table_hbm, pos_hbm, out_hbm,
              idx_v, pos_v, rows_v, *sems):
    gsems, psem, ssem = sems[:NCHUNK], sems[NCHUNK], sems[NCHUNK + 1]
    wid = lax.axis_index("s") * NC + lax.axis_index("c")
    pbase = wid * NPOS

    pcp = pltpu.async_copy(pos_hbm.at[pl.ds(pbase, NPOS)], pos_v, psem)
    idx_cps = [
        pltpu.async_copy(
            x_hbm.at[b, pl.ds(pbase + pofs, n)],
            idx_v.at[k, pl.ds(0, n)],
            gsems[k],
        )
        for k, (b, pofs, n) in enumerate(CHUNKS)
    ]

    gathers = []
    rofs_list = []
    rofs = 0
    for k, (b, pofs, n) in enumerate(CHUNKS):
        idx_cps[k].wait()
        rofs_list.append(rofs)
        gathers.append(
            pltpu.async_copy(
                table_hbm.at[idx_v.at[k, pl.ds(0, n)]],
                rows_v.at[pl.ds(rofs, n)],
                gsems[k],
            )
        )
        rofs += n
    pcp.wait()

    stores = []
    for k, (b, pofs, n) in enumerate(CHUNKS):
        gathers[k].wait()
        rofs = rofs_list[k]

        def row_body(i, carry):
            r = i * 2
            for u in range(2):
                for c in range(DIM // LANES):
                    s = pl.ds(c * LANES, LANES)
                    plsc.addupdate(
                        rows_v.at[rofs + r + u, s], pos_v[pofs + r + u, s]
                    )
            return carry

        lax.fori_loop(0, n // 2, row_body, 0)
        stores.append(
            pltpu.async_copy(
                rows_v.at[pl.ds(rofs, n)],
                out_hbm.at[pl.ds(b * CTX + pbase + pofs, n)],
                ssem,
            )
        )

    for cp in stores:
        cp.wait()


def kernel(x, table, pos_emb):
    out = _embed_sc(x.astype(jnp.int32), table, pos_emb)
    return out.reshape(B, CTX, DIM)
